# Initial kernel scaffold; baseline (speedup 1.0000x reference)
import jax
import jax.numpy as jnp
from jax.experimental import pallas as pl


def kernel(opState, envs):
    raise NotImplementedError("write your pallas kernel here")


# speedup vs baseline: 1.9884x; 1.9884x over previous
import functools

import jax
import jax.numpy as jnp
from jax import lax
from jax.experimental import pallas as pl
from jax.experimental.pallas import tpu as pltpu
from jax.experimental.pallas import tpu_sc as plsc

NC = 2
NS = 16
L = 16
NW = NC * NS

B = 64
N = 4096
G = 200
CELLS = G * G
BPW = B // NW
ITERS = N // L

_mesh = plsc.VectorSubcoreMesh(core_axis_name="c", subcore_axis_name="s")


@functools.partial(
    pl.kernel,
    out_type=jax.ShapeDtypeStruct((NW, L), jnp.float32),
    mesh=_mesh,
    scratch_types=[
        pltpu.VMEM((CELLS,), jnp.float32),
        pltpu.VMEM((2 * N,), jnp.float32),
        pltpu.VMEM((L,), jnp.float32),
    ],
    compiler_params=pltpu.CompilerParams(needs_layout_passes=False),
)
def _collision_sc(op_hbm, env_hbm, out_hbm, emap_v, opst_v, out_v):
    wid = lax.axis_index("s") * NC + lax.axis_index("c")
    lanes = lax.iota(jnp.int32, L)

    acc = jnp.zeros((L,), jnp.float32)
    for j in range(BPW):
        b = wid * BPW + j
        pltpu.sync_copy(env_hbm.at[b], emap_v)
        pltpu.sync_copy(op_hbm.at[b], opst_v)

        def body(i, acc):
            xi = 2 * L * i + 2 * lanes
            px = plsc.load_gather(opst_v, [xi])
            py = plsc.load_gather(opst_v, [xi + 1])

            outr = (px < -9.9) | (px > 9.9) | (py < -9.9) | (py > 9.9)
            cx = jnp.clip(px, -9.9, 9.9)
            cy = jnp.clip(py, -9.9, 9.9)
            ix = ((cx - 0.05) + 10.0) * 10.0
            iy = ((cy - 0.05) + 10.0) * 10.0
            ixi = ix.astype(jnp.int32)
            iyi = iy.astype(jnp.int32)
            dx = (cx - ((ixi.astype(jnp.float32) + 0.5) * 0.1 - 10.0)) * 10.0
            dy = (cy - ((iyi.astype(jnp.float32) + 0.5) * 0.1 - 10.0)) * 10.0

            flat = ixi * G + iyi
            v00 = plsc.load_gather(emap_v, [flat])
            v10 = plsc.load_gather(emap_v, [flat + G])
            v01 = plsc.load_gather(emap_v, [flat + 1])
            v11 = plsc.load_gather(emap_v, [flat + G + 1])

            lo = (1.0 - dx) * v00 + dx * v10
            hi = (1.0 - dx) * v01 + dx * v11
            v0 = (1.0 - dy) * lo + dy * hi
            v0 = jnp.where(outr, -1.0, v0)
            viod = 10.0 * (0.3 - v0)
            pen = jnp.maximum(viod, 0.0)
            return acc + pen * pen

        acc = lax.fori_loop(0, ITERS, body, acc)

    out_v[...] = acc
    pltpu.sync_copy(out_v, out_hbm.at[wid])


def kernel(opState, envs):
    op2d = opState.reshape(B, 2 * N)
    env2d = envs.reshape(B, CELLS)
    partials = _collision_sc(op2d, env2d)
    return jnp.sum(partials) / (B * N)


# speedup vs baseline: 2.1085x; 1.0604x over previous
import functools

import jax
import jax.numpy as jnp
from jax import lax
from jax.experimental import pallas as pl
from jax.experimental.pallas import tpu as pltpu
from jax.experimental.pallas import tpu_sc as plsc

NC = 2
NS = 16
L = 16
NW = NC * NS

B = 64
N = 4096
G = 200
CELLS = G * G
BPW = B // NW
ITERS = N // L

_mesh = plsc.VectorSubcoreMesh(core_axis_name="c", subcore_axis_name="s")


@functools.partial(
    pl.kernel,
    out_type=jax.ShapeDtypeStruct((NW, L), jnp.float32),
    mesh=_mesh,
    scratch_types=[
        pltpu.VMEM((CELLS,), jnp.float32),
        pltpu.VMEM((CELLS,), jnp.float32),
        pltpu.VMEM((2 * N,), jnp.float32),
        pltpu.VMEM((2 * N,), jnp.float32),
        pltpu.VMEM((L,), jnp.float32),
        pltpu.SemaphoreType.DMA,
        pltpu.SemaphoreType.DMA,
    ],
    compiler_params=pltpu.CompilerParams(needs_layout_passes=False),
)
def _collision_sc(op_hbm, env_hbm, out_hbm,
                  emap0_v, emap1_v, opst0_v, opst1_v, out_v, sem0, sem1):
    wid = lax.axis_index("s") * NC + lax.axis_index("c")
    b0 = wid * BPW
    lanes2 = lax.iota(jnp.int32, L) * 2

    cp0e = pltpu.async_copy(env_hbm.at[b0], emap0_v, sem0)
    cp0o = pltpu.async_copy(op_hbm.at[b0], opst0_v, sem0)
    cp1e = pltpu.async_copy(env_hbm.at[b0 + 1], emap1_v, sem1)
    cp1o = pltpu.async_copy(op_hbm.at[b0 + 1], opst1_v, sem1)

    def run_batch(emap_v, opst_v, acc0):
        @plsc.parallel_loop(0, ITERS, unroll=4, carry=acc0)
        def body(i, acc):
            xi = lanes2 + (i * (2 * L))
            px = plsc.load_gather(opst_v, [xi])
            py = plsc.load_gather(opst_v, [xi + 1])

            cx = jnp.clip(px, -9.9, 9.9)
            cy = jnp.clip(py, -9.9, 9.9)
            outr = (cx != px) | (cy != py)
            tx = (cx + 9.95) * 10.0
            ty = (cy + 9.95) * 10.0
            fxi = tx.astype(jnp.int32)
            fyi = ty.astype(jnp.int32)
            dx = tx - fxi.astype(jnp.float32)
            dy = ty - fyi.astype(jnp.float32)

            flat = fxi * G + fyi
            v00 = plsc.load_gather(emap_v, [flat])
            v10 = plsc.load_gather(emap_v, [flat + G])
            v01 = plsc.load_gather(emap_v, [flat + 1])
            v11 = plsc.load_gather(emap_v, [flat + (G + 1)])

            lo = v00 + dx * (v10 - v00)
            hi = v01 + dx * (v11 - v01)
            v0 = lo + dy * (hi - lo)
            v0 = jnp.where(outr, -1.0, v0)
            viod = 3.0 - 10.0 * v0
            pen = jnp.maximum(viod, 0.0)
            return acc + pen * pen

        return body

    cp0e.wait()
    cp0o.wait()
    acc = run_batch(emap0_v, opst0_v, jnp.zeros((L,), jnp.float32))
    cp1e.wait()
    cp1o.wait()
    acc = run_batch(emap1_v, opst1_v, acc)

    out_v[...] = acc
    pltpu.sync_copy(out_v, out_hbm.at[wid])


def kernel(opState, envs):
    op2d = opState.reshape(B, 2 * N)
    env2d = envs.reshape(B, CELLS)
    partials = _collision_sc(op2d, env2d)
    return jnp.sum(partials) / (B * N)


# speedup vs baseline: 2.4988x; 1.1851x over previous
import functools

import jax
import jax.numpy as jnp
from jax import lax
from jax.experimental import pallas as pl
from jax.experimental.pallas import tpu as pltpu
from jax.experimental.pallas import tpu_sc as plsc

NC = 2
NS = 16
L = 16
NW = NC * NS

B = 64
N = 4096
G = 200
BPW = B // NW
ITERS = N // L
OPR = 2 * N // 128

_mesh = plsc.VectorSubcoreMesh(core_axis_name="c", subcore_axis_name="s")


@functools.partial(
    pl.kernel,
    out_type=jax.ShapeDtypeStruct((NW * L,), jnp.float32),
    mesh=_mesh,
    scratch_types=[
        pltpu.VMEM((G, G), jnp.float32),
        pltpu.VMEM((G, G), jnp.float32),
        pltpu.VMEM((OPR, 128), jnp.float32),
        pltpu.VMEM((OPR, 128), jnp.float32),
        pltpu.VMEM((L,), jnp.float32),
        pltpu.SemaphoreType.DMA,
        pltpu.SemaphoreType.DMA,
    ],
    compiler_params=pltpu.CompilerParams(
        needs_layout_passes=False, use_tc_tiling_on_sc=True),
)
def _collision_sc(op_hbm, env_hbm, out_hbm,
                  emap0_v, emap1_v, opst0_v, opst1_v, out_v, sem0, sem1):
    wid = lax.axis_index("s") * NC + lax.axis_index("c")
    b0 = wid * BPW
    lanes = lax.iota(jnp.int32, L)

    cp0e = pltpu.async_copy(env_hbm.at[b0, 0], emap0_v, sem0)
    cp0o = pltpu.async_copy(op_hbm.at[b0], opst0_v, sem0)
    cp1e = pltpu.async_copy(env_hbm.at[b0 + 1, 0], emap1_v, sem1)
    cp1o = pltpu.async_copy(op_hbm.at[b0 + 1], opst1_v, sem1)

    def run_batch(emap_v, opst_v, acc0):
        @plsc.parallel_loop(0, ITERS, unroll=4, carry=acc0)
        def body(i, acc):
            pt = lanes + i * L
            pr = pt >> 6
            pc = (pt << 1) & 127
            px = plsc.load_gather(opst_v, [pr, pc])
            py = plsc.load_gather(opst_v, [pr, pc + 1])

            cx = jnp.clip(px, -9.9, 9.9)
            cy = jnp.clip(py, -9.9, 9.9)
            outr = (cx != px) | (cy != py)
            tx = (cx + 9.95) * 10.0
            ty = (cy + 9.95) * 10.0
            fxi = tx.astype(jnp.int32)
            fyi = ty.astype(jnp.int32)
            dx = tx - fxi.astype(jnp.float32)
            dy = ty - fyi.astype(jnp.float32)

            fxi1 = fxi + 1
            fyi1 = fyi + 1
            v00 = plsc.load_gather(emap_v, [fxi, fyi])
            v10 = plsc.load_gather(emap_v, [fxi1, fyi])
            v01 = plsc.load_gather(emap_v, [fxi, fyi1])
            v11 = plsc.load_gather(emap_v, [fxi1, fyi1])

            lo = v00 + dx * (v10 - v00)
            hi = v01 + dx * (v11 - v01)
            v0 = lo + dy * (hi - lo)
            v0 = jnp.where(outr, -1.0, v0)
            viod = 3.0 - 10.0 * v0
            pen = jnp.maximum(viod, 0.0)
            return acc + pen * pen

        return body

    cp0e.wait()
    cp0o.wait()
    acc = run_batch(emap0_v, opst0_v, jnp.zeros((L,), jnp.float32))
    cp1e.wait()
    cp1o.wait()
    acc = run_batch(emap1_v, opst1_v, acc)

    out_v[...] = acc
    pltpu.sync_copy(out_v, out_hbm.at[pl.ds(wid * L, L)])


def kernel(opState, envs):
    op3d = opState.reshape(B, OPR, 128)
    partials = _collision_sc(op3d, envs)
    return jnp.sum(partials) / (B * N)


# speedup vs baseline: 2.9692x; 1.1883x over previous
import functools

import jax
import jax.numpy as jnp
from jax import lax
from jax.experimental import pallas as pl
from jax.experimental.pallas import tpu as pltpu
from jax.experimental.pallas import tpu_sc as plsc

NC = 2
NS = 16
L = 16
NW = NC * NS

B = 64
N = 4096
G = 200
BPW = B // NW
ITERS = N // L
OPR = 2 * N // 128

_mesh = plsc.VectorSubcoreMesh(core_axis_name="c", subcore_axis_name="s")


@functools.partial(
    pl.kernel,
    out_type=jax.ShapeDtypeStruct((NW * L,), jnp.float32),
    mesh=_mesh,
    scratch_types=[
        pltpu.VMEM((G, G), jnp.float32),
        pltpu.VMEM((G, G), jnp.float32),
        pltpu.VMEM((N // 128, 2, 128), jnp.float32),
        pltpu.VMEM((N // 128, 2, 128), jnp.float32),
        pltpu.VMEM((L,), jnp.float32),
        pltpu.SemaphoreType.DMA,
        pltpu.SemaphoreType.DMA,
    ],
    compiler_params=pltpu.CompilerParams(
        needs_layout_passes=False, use_tc_tiling_on_sc=True),
)
def _collision_sc(op_hbm, env_hbm, out_hbm,
                  emap0_v, emap1_v, opst0_v, opst1_v, out_v, sem0, sem1):
    wid = lax.axis_index("s") * NC + lax.axis_index("c")
    b0 = wid * BPW
    lanes = lax.iota(jnp.int32, L)

    cp0e = pltpu.async_copy(env_hbm.at[b0, 0], emap0_v, sem0)
    cp0o = pltpu.async_copy(op_hbm.at[b0], opst0_v, sem0)
    cp1e = pltpu.async_copy(env_hbm.at[b0 + 1, 0], emap1_v, sem1)
    cp1o = pltpu.async_copy(op_hbm.at[b0 + 1], opst1_v, sem1)

    def run_batch(emap_v, opst_v, acc0):
        @plsc.parallel_loop(0, ITERS, unroll=4, carry=acc0)
        def body(i, acc):
            t = i >> 3
            q = (i & 7) * L
            px = opst_v[t, 0, pl.ds(q, L)]
            py = opst_v[t, 1, pl.ds(q, L)]

            cx = jnp.clip(px, -9.9, 9.9)
            cy = jnp.clip(py, -9.9, 9.9)
            outr = (cx != px) | (cy != py)
            tx = (cx + 9.95) * 10.0
            ty = (cy + 9.95) * 10.0
            fxi = tx.astype(jnp.int32)
            fyi = ty.astype(jnp.int32)
            dx = tx - fxi.astype(jnp.float32)
            dy = ty - fyi.astype(jnp.float32)

            fxi1 = fxi + 1
            fyi1 = fyi + 1
            v00 = plsc.load_gather(emap_v, [fxi, fyi])
            v10 = plsc.load_gather(emap_v, [fxi1, fyi])
            v01 = plsc.load_gather(emap_v, [fxi, fyi1])
            v11 = plsc.load_gather(emap_v, [fxi1, fyi1])

            lo = v00 + dx * (v10 - v00)
            hi = v01 + dx * (v11 - v01)
            v0 = lo + dy * (hi - lo)
            v0 = jnp.where(outr, -1.0, v0)
            viod = 3.0 - 10.0 * v0
            pen = jnp.maximum(viod, 0.0)
            return acc + pen * pen

        return body

    cp0e.wait()
    cp0o.wait()
    acc = run_batch(emap0_v, opst0_v, jnp.zeros((L,), jnp.float32))
    cp1e.wait()
    cp1o.wait()
    acc = run_batch(emap1_v, opst1_v, acc)

    out_v[...] = acc
    pltpu.sync_copy(out_v, out_hbm.at[pl.ds(wid * L, L)])


def kernel(opState, envs):
    op4d = opState.reshape(B, N // 128, 128, 2).transpose(0, 1, 3, 2)
    partials = _collision_sc(op4d, envs)
    return jnp.sum(partials) / (B * N)


# speedup vs baseline: 3.0024x; 1.0112x over previous
import functools

import jax
import jax.numpy as jnp
from jax import lax
from jax.experimental import pallas as pl
from jax.experimental.pallas import tpu as pltpu
from jax.experimental.pallas import tpu_sc as plsc

NC = 2
NS = 16
L = 16
NW = NC * NS

B = 64
N = 4096
G = 200
BPW = B // NW
ITERS = N // L
OPR = 2 * N // 128

_mesh = plsc.VectorSubcoreMesh(core_axis_name="c", subcore_axis_name="s")


@functools.partial(
    pl.kernel,
    out_type=jax.ShapeDtypeStruct((NW * L,), jnp.float32),
    mesh=_mesh,
    scratch_types=[
        pltpu.VMEM((G, G), jnp.float32),
        pltpu.VMEM((G, G), jnp.float32),
        pltpu.VMEM((N // 128, 2, 128), jnp.float32),
        pltpu.VMEM((N // 128, 2, 128), jnp.float32),
        pltpu.VMEM((L,), jnp.float32),
        pltpu.SemaphoreType.DMA,
        pltpu.SemaphoreType.DMA,
    ],
    compiler_params=pltpu.CompilerParams(
        needs_layout_passes=False, use_tc_tiling_on_sc=True),
)
def _collision_sc(op_hbm, env_hbm, out_hbm,
                  emap0_v, emap1_v, opst0_v, opst1_v, out_v, sem0, sem1):
    wid = lax.axis_index("s") * NC + lax.axis_index("c")
    b0 = wid * BPW

    cp0e = pltpu.async_copy(env_hbm.at[b0, 0], emap0_v, sem0)
    cp0o = pltpu.async_copy(op_hbm.at[b0], opst0_v, sem0)
    cp1e = pltpu.async_copy(env_hbm.at[b0 + 1, 0], emap1_v, sem1)
    cp1o = pltpu.async_copy(op_hbm.at[b0 + 1], opst1_v, sem1)

    def run_batch(emap_v, opst_v, acc0):
        @plsc.parallel_loop(0, ITERS, unroll=2, carry=acc0)
        def body(i, acc):
            t = i >> 3
            q = (i & 7) * L
            px = opst_v[t, 0, pl.ds(q, L)]
            py = opst_v[t, 1, pl.ds(q, L)]

            cx = jnp.clip(px, -9.9, 9.9)
            cy = jnp.clip(py, -9.9, 9.9)
            outr = (cx != px) | (cy != py)
            tx = (cx + 9.95) * 10.0
            ty = (cy + 9.95) * 10.0
            fxi = tx.astype(jnp.int32)
            fyi = ty.astype(jnp.int32)
            dx = tx - fxi.astype(jnp.float32)
            dy = ty - fyi.astype(jnp.float32)

            fxi1 = fxi + 1
            fyi1 = fyi + 1
            v00 = plsc.load_gather(emap_v, [fxi, fyi])
            v10 = plsc.load_gather(emap_v, [fxi1, fyi])
            v01 = plsc.load_gather(emap_v, [fxi, fyi1])
            v11 = plsc.load_gather(emap_v, [fxi1, fyi1])

            lo = v00 + dx * (v10 - v00)
            hi = v01 + dx * (v11 - v01)
            v0 = lo + dy * (hi - lo)
            v0 = jnp.where(outr, -1.0, v0)
            viod = 3.0 - 10.0 * v0
            pen = jnp.maximum(viod, 0.0)
            return acc + pen * pen

        return body

    cp0e.wait()
    cp0o.wait()
    acc = run_batch(emap0_v, opst0_v, jnp.zeros((L,), jnp.float32))
    cp1e.wait()
    cp1o.wait()
    acc = run_batch(emap1_v, opst1_v, acc)

    out_v[...] = acc
    pltpu.sync_copy(out_v, out_hbm.at[pl.ds(wid * L, L)])


def kernel(opState, envs):
    op4d = opState.reshape(B, N // 128, 128, 2).transpose(0, 1, 3, 2)
    partials = _collision_sc(op4d, envs)
    return jnp.sum(partials) / (B * N)


# speedup vs baseline: 3.0322x; 1.0099x over previous
import functools

import jax
import jax.numpy as jnp
from jax import lax
from jax.experimental import pallas as pl
from jax.experimental.pallas import tpu as pltpu
from jax.experimental.pallas import tpu_sc as plsc

NC = 2
NS = 16
L = 16
NW = NC * NS

B = 64
N = 4096
G = 200
BPW = B // NW
ITERS = N // L
OPR = 2 * N // 128

_mesh = plsc.VectorSubcoreMesh(core_axis_name="c", subcore_axis_name="s")


@functools.partial(
    pl.kernel,
    out_type=jax.ShapeDtypeStruct((NW * L,), jnp.float32),
    mesh=_mesh,
    scratch_types=[
        pltpu.VMEM((G, G), jnp.float32),
        pltpu.VMEM((G, G), jnp.float32),
        pltpu.VMEM((N // 128, 2, 128), jnp.float32),
        pltpu.VMEM((N // 128, 2, 128), jnp.float32),
        pltpu.VMEM((L,), jnp.float32),
        pltpu.SemaphoreType.DMA,
        pltpu.SemaphoreType.DMA,
    ],
    compiler_params=pltpu.CompilerParams(
        needs_layout_passes=False, use_tc_tiling_on_sc=True),
)
def _collision_sc(op_hbm, env_hbm, out_hbm,
                  emap0_v, emap1_v, opst0_v, opst1_v, out_v, sem0, sem1):
    wid = lax.axis_index("s") * NC + lax.axis_index("c")
    b0 = wid * BPW

    cp0e = pltpu.async_copy(env_hbm.at[b0, 0], emap0_v, sem0)
    cp0o = pltpu.async_copy(op_hbm.at[b0], opst0_v, sem0)
    cp1e = pltpu.async_copy(env_hbm.at[b0 + 1, 0], emap1_v, sem1)
    cp1o = pltpu.async_copy(op_hbm.at[b0 + 1], opst1_v, sem1)

    def run_batch(emap_v, opst_v, acc0):
        @plsc.parallel_loop(0, ITERS, unroll=2, carry=acc0)
        def body(i, acc):
            t = i >> 3
            q = (i & 7) * L
            px = opst_v[t, 0, pl.ds(q, L)]
            py = opst_v[t, 1, pl.ds(q, L)]

            cx = jnp.clip(px, -9.9, 9.9)
            cy = jnp.clip(py, -9.9, 9.9)
            outr = jnp.maximum(jnp.abs(px), jnp.abs(py)) > 9.9
            tx = (cx + 9.95) * 10.0
            ty = (cy + 9.95) * 10.0
            fxi = tx.astype(jnp.int32)
            fyi = ty.astype(jnp.int32)
            dx = tx - fxi.astype(jnp.float32)
            dy = ty - fyi.astype(jnp.float32)

            fxi1 = fxi + 1
            fyi1 = fyi + 1
            v00 = plsc.load_gather(emap_v, [fxi, fyi])
            v10 = plsc.load_gather(emap_v, [fxi1, fyi])
            v01 = plsc.load_gather(emap_v, [fxi, fyi1])
            v11 = plsc.load_gather(emap_v, [fxi1, fyi1])

            lo = v00 + dx * (v10 - v00)
            hi = v01 + dx * (v11 - v01)
            v0 = lo + dy * (hi - lo)
            v0 = jnp.where(outr, -1.0, v0)
            pen = jnp.maximum(0.3 - v0, 0.0)
            return acc + pen * pen

        return body

    cp0e.wait()
    cp0o.wait()
    acc = run_batch(emap0_v, opst0_v, jnp.zeros((L,), jnp.float32))
    cp1e.wait()
    cp1o.wait()
    acc = run_batch(emap1_v, opst1_v, acc)

    out_v[...] = acc * 100.0
    pltpu.sync_copy(out_v, out_hbm.at[pl.ds(wid * L, L)])


def kernel(opState, envs):
    op4d = opState.reshape(B, N // 128, 128, 2).transpose(0, 1, 3, 2)
    partials = _collision_sc(op4d, envs)
    return jnp.sum(partials) / (B * N)
